# baseline (speedup 1.0000x reference)
import functools

import jax
import jax.numpy as jnp
from jax import lax
from jax.experimental import pallas as pl
from jax.experimental.pallas import tpu as pltpu
from jax.experimental.pallas import tpu_sc as plsc

N = 10000
NPAD = 10240
NACC = 10240
DUMMY = 10000
E = 160000
EPAD = 163840
ECH = EPAD // 128
NL = 4000
P = 8192
D_IN = 2813
KPAD = 2816
RB = 2048
NRB = NPAD // RB

_RPT = NACC // 16


def _mesh():
    return plsc.VectorSubcoreMesh(core_axis_name="c", subcore_axis_name="s",
                                  num_cores=2, num_subcores=16)


def _make_deg():
    @functools.partial(
        pl.kernel,
        out_type=jax.ShapeDtypeStruct((2, NPAD, 128), jnp.float32),
        mesh=_mesh(),
        scratch_types=[
            pltpu.VMEM((40, 128), jnp.int32),
            pltpu.VMEM((128, 128), jnp.float32),
            pltpu.VMEM_SHARED((NACC, 128), jnp.float32),
        ],
    )
    def deg_k(dstp_hbm, zeros_hbm, ones_hbm, out_hbm, idx_v, ones_v, acc_sh):
        cid = lax.axis_index("c")
        sid = lax.axis_index("s")
        g = cid * 16 + sid
        r0 = sid * _RPT
        pltpu.sync_copy(zeros_hbm.at[pl.ds(r0, _RPT)], acc_sh.at[pl.ds(r0, _RPT)])
        pltpu.sync_copy(ones_hbm, ones_v)
        pltpu.sync_copy(dstp_hbm.at[pl.ds(g * 40, 40)], idx_v)
        plsc.subcore_barrier()

        def body(i, c):
            pltpu.sync_copy(ones_v, acc_sh.at[idx_v.at[i]], add=True)
            return c

        lax.fori_loop(0, 40, body, 0)
        plsc.subcore_barrier()
        pltpu.sync_copy(acc_sh.at[pl.ds(r0, _RPT)],
                        out_hbm.at[cid, pl.ds(r0, _RPT)])

    return deg_k


def _make_agg(ncb):
    ch64 = EPAD // 16 // 64
    nb = 4
    nq = 4
    nh = ch64 // nq

    @functools.partial(
        pl.kernel,
        out_type=jax.ShapeDtypeStruct((ncb, NPAD, 128), jnp.float32),
        mesh=_mesh(),
        scratch_types=(
            [pltpu.VMEM((nh, 64), jnp.int32),
             pltpu.VMEM((nh, 64), jnp.int32)]
            + [pltpu.VMEM((64, 128), jnp.float32) for _ in range(nb)]
            + [pltpu.VMEM_SHARED((NACC, 128), jnp.float32)]
            + [pltpu.SemaphoreType.DMA for _ in range(2 * nb)]
        ),
    )
    def agg_k(yf_hbm, srccb_hbm, dstp_hbm, out_hbm, isrc_v, idst_v, *rest):
        rows = rest[:nb]
        acc_sh = rest[nb]
        gsem = rest[nb + 1:2 * nb + 1]
        ssem = rest[2 * nb + 1:3 * nb + 1]
        cid = lax.axis_index("c")
        sid = lax.axis_index("s")
        r0 = sid * _RPT
        for k in range(ncb // 2):
            cb = 2 * k + cid
            pltpu.sync_copy(yf_hbm.at[pl.ds(cb * NPAD + r0, _RPT)],
                            acc_sh.at[pl.ds(r0, _RPT)])
            plsc.subcore_barrier()
            for h in range(nq):
                base = sid * ch64 + h * nh
                pltpu.sync_copy(srccb_hbm.at[cb, pl.ds(base, nh)], isrc_v)
                pltpu.sync_copy(dstp_hbm.at[pl.ds(base, nh)], idst_v)
                for b in range(nb):
                    pltpu.async_copy(yf_hbm.at[isrc_v.at[b]], rows[b], gsem[b])

                @pl.loop(0, nh, step=nb)
                def _(i):
                    for b in range(nb):
                        ch = i + b
                        pltpu.make_async_copy(yf_hbm.at[isrc_v.at[ch]],
                                              rows[b], gsem[b]).wait()
                        pltpu.async_copy(rows[b], acc_sh.at[idst_v.at[ch]],
                                         ssem[b], add=True)

                        @pl.when(ch + nb < nh)
                        def _():
                            pltpu.make_async_copy(
                                rows[b], acc_sh.at[idst_v.at[ch]],
                                ssem[b]).wait()
                            pltpu.async_copy(yf_hbm.at[isrc_v.at[ch + nb]],
                                             rows[b], gsem[b])

                for b in range(nb):
                    pltpu.make_async_copy(rows[b],
                                          acc_sh.at[idst_v.at[nh - nb + b]],
                                          ssem[b]).wait()
            plsc.subcore_barrier()
            pltpu.sync_copy(acc_sh.at[pl.ds(r0, _RPT)],
                            out_hbm.at[cb, pl.ds(r0, _RPT)])

    return agg_k


def _make_pair_gather():
    @functools.partial(
        pl.kernel,
        out_type=jax.ShapeDtypeStruct((8, P, 128), jnp.float32),
        mesh=_mesh(),
        scratch_types=[
            pltpu.VMEM((8, P // 128, 128), jnp.int32),
            pltpu.VMEM((128, 128), jnp.float32),
        ],
    )
    def pair_k(x2f_hbm, idx_hbm, out_hbm, idx_v, rows_v):
        cid = lax.axis_index("c")
        sid = lax.axis_index("s")
        w = sid * 2 + cid
        pltpu.sync_copy(idx_hbm, idx_v)
        for c in range(2):
            row = w * 2 + c
            r0 = row * 128
            for b in range(8):
                pltpu.sync_copy(x2f_hbm.at[idx_v.at[b, row]], rows_v)
                pltpu.sync_copy(rows_v, out_hbm.at[b, pl.ds(r0, 128)])

    return pair_k


_SC = {}


def _sc_kernels():
    if not _SC:
        _SC["deg"] = _make_deg()
        _SC["agg8"] = _make_agg(8)
        _SC["agg4"] = _make_agg(4)
        _SC["pair"] = _make_pair_gather()
    return _SC


def _dinv(deg_ref):
    d = deg_ref[0, :, 0:1] + deg_ref[1, :, 0:1] + 1.0
    return lax.rsqrt(d)


def _deg_spec():
    return pl.BlockSpec((2, RB, 128), lambda i, *_: (0, i, 0))


def _ymm1_body(x_ref, w_ref, deg_ref, o_ref):
    acc = jnp.dot(x_ref[...], w_ref[...], preferred_element_type=jnp.float32)
    o_ref[0] = acc * _dinv(deg_ref)


def _rmm1_body(x_ref, w_ref, b_ref, o_ref):
    acc = jnp.dot(x_ref[...], w_ref[...], preferred_element_type=jnp.float32)
    o_ref[0] = acc + b_ref[...]


def _x1_blocks(r1_ref, agg_ref, b1_ref, dinv):
    for cb in range(8):
        yield jax.nn.relu(r1_ref[cb] + b1_ref[cb] + dinv * agg_ref[cb])


def _ymm2_body(r1_ref, agg_ref, deg_ref, b1_ref, w_ref, o_ref):
    dinv = _dinv(deg_ref)
    acc = jnp.zeros((RB, 512), jnp.float32)
    for cb, x1 in enumerate(_x1_blocks(r1_ref, agg_ref, b1_ref, dinv)):
        acc += jnp.dot(x1, w_ref[cb], preferred_element_type=jnp.float32)
    y2 = acc * dinv
    for cb in range(4):
        o_ref[cb] = y2[:, cb * 128:(cb + 1) * 128]


def _rmm2_body(r1_ref, agg_ref, deg_ref, b1_ref, w_ref, b2_ref, o_ref):
    dinv = _dinv(deg_ref)
    acc = jnp.zeros((RB, 512), jnp.float32)
    for cb, x1 in enumerate(_x1_blocks(r1_ref, agg_ref, b1_ref, dinv)):
        acc += jnp.dot(x1, w_ref[cb], preferred_element_type=jnp.float32)
    for cb in range(4):
        o_ref[cb] = acc[:, cb * 128:(cb + 1) * 128] + b2_ref[cb]


def _x2_body(r2_ref, agg_ref, deg_ref, b2_ref, o_ref):
    dinv = _dinv(deg_ref)
    for cb in range(4):
        o_ref[cb] = jax.nn.relu(r2_ref[cb] + b2_ref[cb] + dinv * agg_ref[cb])


def _head_body(f_ref, w1_ref, b1_ref, w2_ref, b2_ref, w3_ref, b3_ref, o_ref):
    acc = jnp.zeros((1024, 128), jnp.float32)
    for cb in range(8):
        acc += jnp.dot(f_ref[cb], w1_ref[cb], preferred_element_type=jnp.float32)
    h3 = jax.nn.relu(acc + b1_ref[...])
    h4 = jax.nn.relu(jnp.dot(h3, w2_ref[...],
                             preferred_element_type=jnp.float32) + b2_ref[...])
    o_ref[...] = jax.nn.sigmoid(jnp.dot(h4, w3_ref[...],
                                        preferred_element_type=jnp.float32)
                                + b3_ref[...])


_ymm1 = pl.pallas_call(
    _ymm1_body,
    grid=(NRB, 8),
    in_specs=[
        pl.BlockSpec((RB, KPAD), lambda i, j: (i, 0)),
        pl.BlockSpec((KPAD, 128), lambda i, j: (0, j)),
        _deg_spec(),
    ],
    out_specs=pl.BlockSpec((1, RB, 128), lambda i, j: (j, i, 0)),
    out_shape=jax.ShapeDtypeStruct((8, NPAD, 128), jnp.float32),
)

_rmm1 = pl.pallas_call(
    _rmm1_body,
    grid=(NRB, 8),
    in_specs=[
        pl.BlockSpec((RB, KPAD), lambda i, j: (i, 0)),
        pl.BlockSpec((KPAD, 128), lambda i, j: (0, j)),
        pl.BlockSpec((1, 128), lambda i, j: (0, j)),
    ],
    out_specs=pl.BlockSpec((1, RB, 128), lambda i, j: (j, i, 0)),
    out_shape=jax.ShapeDtypeStruct((8, NPAD, 128), jnp.float32),
)

_b1_spec = pl.BlockSpec((8, 1, 128), lambda i: (0, 0, 0))
_blk8_spec = pl.BlockSpec((8, RB, 128), lambda i: (0, i, 0))
_blk4_spec = pl.BlockSpec((4, RB, 128), lambda i: (0, i, 0))
_w2_spec = pl.BlockSpec((8, 128, 512), lambda i: (0, 0, 0))
_out4_spec = pl.BlockSpec((4, RB, 128), lambda i: (0, i, 0))

_ymm2 = pl.pallas_call(
    _ymm2_body,
    grid=(NRB,),
    in_specs=[_blk8_spec, _blk8_spec, _deg_spec(), _b1_spec, _w2_spec],
    out_specs=_out4_spec,
    out_shape=jax.ShapeDtypeStruct((4, NPAD, 128), jnp.float32),
)

_rmm2 = pl.pallas_call(
    _rmm2_body,
    grid=(NRB,),
    in_specs=[_blk8_spec, _blk8_spec, _deg_spec(), _b1_spec, _w2_spec,
              pl.BlockSpec((4, 1, 128), lambda i: (0, 0, 0))],
    out_specs=_out4_spec,
    out_shape=jax.ShapeDtypeStruct((4, NPAD, 128), jnp.float32),
)

_x2k = pl.pallas_call(
    _x2_body,
    grid=(NRB,),
    in_specs=[_blk4_spec, _blk4_spec, _deg_spec(),
              pl.BlockSpec((4, 1, 128), lambda i: (0, 0, 0))],
    out_specs=_out4_spec,
    out_shape=jax.ShapeDtypeStruct((4, NPAD, 128), jnp.float32),
)

_head = pl.pallas_call(
    _head_body,
    grid=(P // 1024,),
    in_specs=[
        pl.BlockSpec((8, 1024, 128), lambda i: (0, i, 0)),
        pl.BlockSpec((8, 128, 128), lambda i: (0, 0, 0)),
        pl.BlockSpec((1, 128), lambda i: (0, 0)),
        pl.BlockSpec((128, 128), lambda i: (0, 0)),
        pl.BlockSpec((1, 128), lambda i: (0, 0)),
        pl.BlockSpec((128, 128), lambda i: (0, 0)),
        pl.BlockSpec((1, 128), lambda i: (0, 0)),
    ],
    out_specs=pl.BlockSpec((1024, 128), lambda i: (i, 0)),
    out_shape=jax.ShapeDtypeStruct((P, 128), jnp.float32),
)


def kernel(x, a, sample_train, W1, b1, W2, b2, f1W, f1b, f2W, f2b,
           fc1W, fc1b, fc2W, fc2b, fc3W, fc3b):
    f32 = jnp.float32
    sc = _sc_kernels()
    src, dst = a[0], a[1]
    pe = EPAD - E
    srcp = jnp.concatenate([src, jnp.zeros((pe,), jnp.int32)]).reshape(ECH, 128)
    dstp = jnp.concatenate([dst, jnp.full((pe,), DUMMY, jnp.int32)]).reshape(ECH, 128)
    srcp64 = srcp.reshape(2 * ECH, 64)
    dstp64 = dstp.reshape(2 * ECH, 64)

    degp = sc["deg"](dstp, jnp.zeros((NACC, 128), f32),
                     jnp.ones((128, 128), f32))

    bf16 = jnp.bfloat16
    xp = jnp.pad(x, ((0, NPAD - N), (0, KPAD - D_IN))).astype(bf16)
    W1p = jnp.pad(W1, ((0, KPAD - D_IN), (0, 0))).astype(bf16)
    f1Wp = jnp.pad(f1W, ((0, KPAD - D_IN), (0, 0))).astype(bf16)
    y1b = _ymm1(xp, W1p, degp)
    r1b = _rmm1(xp, f1Wp, f1b.reshape(1, 1024))

    off8 = (jnp.arange(8, dtype=jnp.int32) * NPAD)[:, None, None]
    agg1b = sc["agg8"](y1b.reshape(8 * NPAD, 128), srcp64[None] + off8, dstp64)

    b1b = b1.reshape(8, 1, 128)
    W2b = jnp.pad(W2, ((0, 0), (0, 112))).reshape(8, 128, 512)
    f2Wb = jnp.pad(f2W, ((0, 0), (0, 112))).reshape(8, 128, 512)
    f2bb = jnp.pad(f2b, (0, 112)).reshape(4, 1, 128)
    y2b = _ymm2(r1b, agg1b, degp, b1b, W2b)
    r2b = _rmm2(r1b, agg1b, degp, b1b, f2Wb, f2bb)

    off4 = (jnp.arange(4, dtype=jnp.int32) * NPAD)[:, None, None]
    agg2b = sc["agg4"](y2b.reshape(4 * NPAD, 128), srcp64[None] + off4, dstp64)

    b2b = jnp.pad(b2, (0, 112)).reshape(4, 1, 128)
    x2b = _x2k(r2b, agg2b, degp, b2b)

    slp = sample_train[:, 0].reshape(P // 128, 128)
    srp = (NL + sample_train[:, 1]).reshape(P // 128, 128)
    idx_all = jnp.concatenate([slp[None] + off4, srp[None] + off4], axis=0)
    feat = sc["pair"](x2b.reshape(4 * NPAD, 128), idx_all)

    fc1Wb = jnp.zeros((1024, 128), f32)
    fc1Wb = fc1Wb.at[0:400].set(fc1W[0:400]).at[512:912].set(fc1W[400:800])
    fc1Wb = fc1Wb.reshape(8, 128, 128)
    fc2Wp = jnp.pad(fc2W, ((0, 0), (0, 96)))
    fc2bp = jnp.pad(fc2b, (0, 96)).reshape(1, 128)
    fc3Wp = jnp.zeros((128, 128), f32).at[0:32, 0:1].set(fc3W)
    b3r = jnp.broadcast_to(fc3b.reshape(1, 1), (1, 128))
    out_full = _head(feat, fc1Wb, fc1b.reshape(1, 128), fc2Wp, fc2bp, fc3Wp, b3r)
    return out_full[:, :1]


# speedup vs baseline: 1.0264x; 1.0264x over previous
import functools

import jax
import jax.numpy as jnp
from jax import lax
from jax.experimental import pallas as pl
from jax.experimental.pallas import tpu as pltpu
from jax.experimental.pallas import tpu_sc as plsc

N = 10000
NPAD = 10240
NACC = 10240
DUMMY = 10000
E = 160000
EPAD = 163840
ECH = EPAD // 128
NL = 4000
P = 8192
D_IN = 2813
KPAD = 2816
RB = 2048
NRB = NPAD // RB

_RPT = NACC // 16


def _mesh():
    return plsc.VectorSubcoreMesh(core_axis_name="c", subcore_axis_name="s",
                                  num_cores=2, num_subcores=16)


def _make_deg():
    @functools.partial(
        pl.kernel,
        out_type=jax.ShapeDtypeStruct((2, NPAD, 128), jnp.float32),
        mesh=_mesh(),
        scratch_types=[
            pltpu.VMEM((40, 128), jnp.int32),
            pltpu.VMEM((128, 128), jnp.float32),
            pltpu.VMEM_SHARED((NACC, 128), jnp.float32),
        ],
    )
    def deg_k(dstp_hbm, zeros_hbm, ones_hbm, out_hbm, idx_v, ones_v, acc_sh):
        cid = lax.axis_index("c")
        sid = lax.axis_index("s")
        g = cid * 16 + sid
        r0 = sid * _RPT
        pltpu.sync_copy(zeros_hbm.at[pl.ds(r0, _RPT)], acc_sh.at[pl.ds(r0, _RPT)])
        pltpu.sync_copy(ones_hbm, ones_v)
        pltpu.sync_copy(dstp_hbm.at[pl.ds(g * 40, 40)], idx_v)
        plsc.subcore_barrier()

        def body(i, c):
            pltpu.sync_copy(ones_v, acc_sh.at[idx_v.at[i]], add=True)
            return c

        lax.fori_loop(0, 40, body, 0)
        plsc.subcore_barrier()
        pltpu.sync_copy(acc_sh.at[pl.ds(r0, _RPT)],
                        out_hbm.at[cid, pl.ds(r0, _RPT)])

    return deg_k


def _make_agg(ncb):
    ch64 = EPAD // 16 // 64
    nb = 4
    nq = 4
    nh = ch64 // nq

    @functools.partial(
        pl.kernel,
        out_type=jax.ShapeDtypeStruct((ncb, NPAD, 128), jnp.float32),
        mesh=_mesh(),
        scratch_types=(
            [pltpu.VMEM((nh, 64), jnp.int32),
             pltpu.VMEM((nh, 64), jnp.int32)]
            + [pltpu.VMEM((64, 128), jnp.float32) for _ in range(nb)]
            + [pltpu.VMEM_SHARED((NACC, 128), jnp.float32)]
            + [pltpu.SemaphoreType.DMA for _ in range(2 * nb)]
        ),
    )
    def agg_k(yf_hbm, srccb_hbm, dstp_hbm, out_hbm, isrc_v, idst_v, *rest):
        rows = rest[:nb]
        acc_sh = rest[nb]
        gsem = rest[nb + 1:2 * nb + 1]
        ssem = rest[2 * nb + 1:3 * nb + 1]
        cid = lax.axis_index("c")
        sid = lax.axis_index("s")
        r0 = sid * _RPT
        for k in range(ncb // 2):
            cb = 2 * k + cid
            pltpu.sync_copy(yf_hbm.at[pl.ds(cb * NPAD + r0, _RPT)],
                            acc_sh.at[pl.ds(r0, _RPT)])
            plsc.subcore_barrier()
            for h in range(nq):
                base = sid * ch64 + h * nh
                pltpu.sync_copy(srccb_hbm.at[cb, pl.ds(base, nh)], isrc_v)
                pltpu.sync_copy(dstp_hbm.at[pl.ds(base, nh)], idst_v)
                for b in range(nb):
                    pltpu.async_copy(yf_hbm.at[isrc_v.at[b]], rows[b], gsem[b])

                @pl.loop(0, nh, step=nb)
                def _(i):
                    for b in range(nb):
                        ch = i + b
                        pltpu.make_async_copy(yf_hbm.at[isrc_v.at[ch]],
                                              rows[b], gsem[b]).wait()
                        pltpu.async_copy(rows[b], acc_sh.at[idst_v.at[ch]],
                                         ssem[b], add=True)

                        @pl.when(ch + nb < nh)
                        def _():
                            pltpu.make_async_copy(
                                rows[b], acc_sh.at[idst_v.at[ch]],
                                ssem[b]).wait()
                            pltpu.async_copy(yf_hbm.at[isrc_v.at[ch + nb]],
                                             rows[b], gsem[b])

                for b in range(nb):
                    pltpu.make_async_copy(rows[b],
                                          acc_sh.at[idst_v.at[nh - nb + b]],
                                          ssem[b]).wait()
            plsc.subcore_barrier()
            pltpu.sync_copy(acc_sh.at[pl.ds(r0, _RPT)],
                            out_hbm.at[cb, pl.ds(r0, _RPT)])

    return agg_k


def _make_pair_gather():
    @functools.partial(
        pl.kernel,
        out_type=jax.ShapeDtypeStruct((8, P, 128), jnp.float32),
        mesh=_mesh(),
        scratch_types=[
            pltpu.VMEM((8, P // 128, 128), jnp.int32),
            pltpu.VMEM((128, 128), jnp.float32),
        ],
    )
    def pair_k(x2f_hbm, idx_hbm, out_hbm, idx_v, rows_v):
        cid = lax.axis_index("c")
        sid = lax.axis_index("s")
        w = sid * 2 + cid
        pltpu.sync_copy(idx_hbm, idx_v)
        for c in range(2):
            row = w * 2 + c
            r0 = row * 128
            for b in range(8):
                pltpu.sync_copy(x2f_hbm.at[idx_v.at[b, row]], rows_v)
                pltpu.sync_copy(rows_v, out_hbm.at[b, pl.ds(r0, 128)])

    return pair_k


_SC = {}


def _sc_kernels():
    if not _SC:
        _SC["deg"] = _make_deg()
        _SC["agg8"] = _make_agg(8)
        _SC["agg4"] = _make_agg(4)
        _SC["pair"] = _make_pair_gather()
    return _SC


def _dinv(deg_ref):
    d = deg_ref[0, :, 0:1] + deg_ref[1, :, 0:1] + 1.0
    return lax.rsqrt(d)


def _deg_spec():
    return pl.BlockSpec((2, RB, 128), lambda i, *_: (0, i, 0))


def _ymm1_body(x_ref, w_ref, deg_ref, o_ref):
    acc = jnp.dot(x_ref[...], w_ref[...], preferred_element_type=jnp.float32)
    o_ref[0] = acc * _dinv(deg_ref)


def _rmm1_body(x_ref, w_ref, b_ref, o_ref):
    acc = jnp.dot(x_ref[...], w_ref[...], preferred_element_type=jnp.float32)
    o_ref[0] = acc + b_ref[...]


def _x1_blocks(r1_ref, agg_ref, b1_ref, dinv):
    for cb in range(8):
        yield jax.nn.relu(r1_ref[cb] + b1_ref[cb] + dinv * agg_ref[cb])


def _ymm2_body(r1_ref, agg_ref, deg_ref, b1_ref, w_ref, o_ref):
    dinv = _dinv(deg_ref)
    acc = jnp.zeros((RB, 512), jnp.float32)
    for cb, x1 in enumerate(_x1_blocks(r1_ref, agg_ref, b1_ref, dinv)):
        acc += jnp.dot(x1, w_ref[cb], preferred_element_type=jnp.float32)
    y2 = acc * dinv
    for cb in range(4):
        o_ref[cb] = y2[:, cb * 128:(cb + 1) * 128]


def _rmm2_body(r1_ref, agg_ref, deg_ref, b1_ref, w_ref, b2_ref, o_ref):
    dinv = _dinv(deg_ref)
    acc = jnp.zeros((RB, 512), jnp.float32)
    for cb, x1 in enumerate(_x1_blocks(r1_ref, agg_ref, b1_ref, dinv)):
        acc += jnp.dot(x1, w_ref[cb], preferred_element_type=jnp.float32)
    for cb in range(4):
        o_ref[cb] = acc[:, cb * 128:(cb + 1) * 128] + b2_ref[cb]


def _x2_body(r2_ref, agg_ref, deg_ref, b2_ref, o_ref):
    dinv = _dinv(deg_ref)
    for cb in range(4):
        o_ref[cb] = jax.nn.relu(r2_ref[cb] + b2_ref[cb] + dinv * agg_ref[cb])


def _head_body(f_ref, w1_ref, b1_ref, w2_ref, b2_ref, w3_ref, b3_ref, o_ref):
    acc = jnp.zeros((1024, 128), jnp.float32)
    for cb in range(8):
        acc += jnp.dot(f_ref[cb], w1_ref[cb], preferred_element_type=jnp.float32)
    h3 = jax.nn.relu(acc + b1_ref[...])
    h4 = jax.nn.relu(jnp.dot(h3, w2_ref[...],
                             preferred_element_type=jnp.float32) + b2_ref[...])
    o_ref[...] = jax.nn.sigmoid(jnp.dot(h4, w3_ref[...],
                                        preferred_element_type=jnp.float32)
                                + b3_ref[...])


_ymm1 = pl.pallas_call(
    _ymm1_body,
    grid=(NRB, 8),
    in_specs=[
        pl.BlockSpec((RB, KPAD), lambda i, j: (i, 0)),
        pl.BlockSpec((KPAD, 128), lambda i, j: (0, j)),
        _deg_spec(),
    ],
    out_specs=pl.BlockSpec((1, RB, 128), lambda i, j: (j, i, 0)),
    out_shape=jax.ShapeDtypeStruct((8, NPAD, 128), jnp.float32),
)

_rmm1 = pl.pallas_call(
    _rmm1_body,
    grid=(NRB, 8),
    in_specs=[
        pl.BlockSpec((RB, KPAD), lambda i, j: (i, 0)),
        pl.BlockSpec((KPAD, 128), lambda i, j: (0, j)),
        pl.BlockSpec((1, 128), lambda i, j: (0, j)),
    ],
    out_specs=pl.BlockSpec((1, RB, 128), lambda i, j: (j, i, 0)),
    out_shape=jax.ShapeDtypeStruct((8, NPAD, 128), jnp.float32),
)

_b1_spec = pl.BlockSpec((8, 1, 128), lambda i: (0, 0, 0))
_blk8_spec = pl.BlockSpec((8, RB, 128), lambda i: (0, i, 0))
_blk4_spec = pl.BlockSpec((4, RB, 128), lambda i: (0, i, 0))
_w2_spec = pl.BlockSpec((8, 128, 512), lambda i: (0, 0, 0))
_out4_spec = pl.BlockSpec((4, RB, 128), lambda i: (0, i, 0))

_ymm2 = pl.pallas_call(
    _ymm2_body,
    grid=(NRB,),
    in_specs=[_blk8_spec, _blk8_spec, _deg_spec(), _b1_spec, _w2_spec],
    out_specs=_out4_spec,
    out_shape=jax.ShapeDtypeStruct((4, NPAD, 128), jnp.float32),
)

_rmm2 = pl.pallas_call(
    _rmm2_body,
    grid=(NRB,),
    in_specs=[_blk8_spec, _blk8_spec, _deg_spec(), _b1_spec, _w2_spec,
              pl.BlockSpec((4, 1, 128), lambda i: (0, 0, 0))],
    out_specs=_out4_spec,
    out_shape=jax.ShapeDtypeStruct((4, NPAD, 128), jnp.float32),
)

_x2k = pl.pallas_call(
    _x2_body,
    grid=(NRB,),
    in_specs=[_blk4_spec, _blk4_spec, _deg_spec(),
              pl.BlockSpec((4, 1, 128), lambda i: (0, 0, 0))],
    out_specs=_out4_spec,
    out_shape=jax.ShapeDtypeStruct((4, NPAD, 128), jnp.float32),
)

_head = pl.pallas_call(
    _head_body,
    grid=(P // 1024,),
    in_specs=[
        pl.BlockSpec((8, 1024, 128), lambda i: (0, i, 0)),
        pl.BlockSpec((8, 128, 128), lambda i: (0, 0, 0)),
        pl.BlockSpec((1, 128), lambda i: (0, 0)),
        pl.BlockSpec((128, 128), lambda i: (0, 0)),
        pl.BlockSpec((1, 128), lambda i: (0, 0)),
        pl.BlockSpec((128, 128), lambda i: (0, 0)),
        pl.BlockSpec((1, 128), lambda i: (0, 0)),
    ],
    out_specs=pl.BlockSpec((1024, 128), lambda i: (i, 0)),
    out_shape=jax.ShapeDtypeStruct((P, 128), jnp.float32),
)


def kernel(x, a, sample_train, W1, b1, W2, b2, f1W, f1b, f2W, f2b,
           fc1W, fc1b, fc2W, fc2b, fc3W, fc3b):
    f32 = jnp.float32
    sc = _sc_kernels()
    src, dst = a[0], a[1]
    pe = EPAD - E
    srcp = jnp.concatenate([src, jnp.zeros((pe,), jnp.int32)]).reshape(ECH, 128)
    dstp = jnp.concatenate([dst, jnp.full((pe,), DUMMY, jnp.int32)]).reshape(ECH, 128)
    srcp64 = srcp.reshape(2 * ECH, 64)
    dstp64 = dstp.reshape(2 * ECH, 64)

    degp = sc["deg"](dstp, jnp.zeros((NACC, 128), f32),
                     jnp.ones((128, 128), f32))

    xp = jnp.pad(x, ((0, NPAD - N), (0, KPAD - D_IN)))
    W1p = jnp.pad(W1, ((0, KPAD - D_IN), (0, 0)))
    f1Wp = jnp.pad(f1W, ((0, KPAD - D_IN), (0, 0)))
    y1b = _ymm1(xp, W1p, degp)
    r1b = _rmm1(xp, f1Wp, f1b.reshape(1, 1024))

    off8 = (jnp.arange(8, dtype=jnp.int32) * NPAD)[:, None, None]
    agg1b = sc["agg8"](y1b.reshape(8 * NPAD, 128), srcp64[None] + off8, dstp64)

    b1b = b1.reshape(8, 1, 128)
    W2b = jnp.pad(W2, ((0, 0), (0, 112))).reshape(8, 128, 512)
    f2Wb = jnp.pad(f2W, ((0, 0), (0, 112))).reshape(8, 128, 512)
    f2bb = jnp.pad(f2b, (0, 112)).reshape(4, 1, 128)
    y2b = _ymm2(r1b, agg1b, degp, b1b, W2b)
    r2b = _rmm2(r1b, agg1b, degp, b1b, f2Wb, f2bb)

    off4 = (jnp.arange(4, dtype=jnp.int32) * NPAD)[:, None, None]
    agg2b = sc["agg4"](y2b.reshape(4 * NPAD, 128), srcp64[None] + off4, dstp64)

    b2b = jnp.pad(b2, (0, 112)).reshape(4, 1, 128)
    x2b = _x2k(r2b, agg2b, degp, b2b)

    slp = sample_train[:, 0].reshape(P // 128, 128)
    srp = (NL + sample_train[:, 1]).reshape(P // 128, 128)
    idx_all = jnp.concatenate([slp[None] + off4, srp[None] + off4], axis=0)
    feat = sc["pair"](x2b.reshape(4 * NPAD, 128), idx_all)

    fc1Wb = jnp.zeros((1024, 128), f32)
    fc1Wb = fc1Wb.at[0:400].set(fc1W[0:400]).at[512:912].set(fc1W[400:800])
    fc1Wb = fc1Wb.reshape(8, 128, 128)
    fc2Wp = jnp.pad(fc2W, ((0, 0), (0, 96)))
    fc2bp = jnp.pad(fc2b, (0, 96)).reshape(1, 128)
    fc3Wp = jnp.zeros((128, 128), f32).at[0:32, 0:1].set(fc3W)
    b3r = jnp.broadcast_to(fc3b.reshape(1, 1), (1, 128))
    out_full = _head(feat, fc1Wb, fc1b.reshape(1, 128), fc2Wp, fc2bp, fc3Wp, b3r)
    return out_full[:, :1]
